# baseline (speedup 1.0000x reference)
import jax
import jax.numpy as jnp
from jax.experimental import pallas as pl
from jax.experimental.pallas import tpu as pltpu

N_TOKENS = 32768
D_EMBED = 768
N_EXPERTS = 64
CHUNK = 512
NBUF = 8
NCHUNK = N_TOKENS // CHUNK


def _router_body(x_hbm, wt_ref, b_ref, scores_ref, idx_ref, buf, sem):
    wt = wt_ref[...]
    bias = b_ref[...]

    def start(j):
        slot = j % NBUF
        pltpu.make_async_copy(
            x_hbm.at[pl.ds(j * CHUNK, CHUNK), :], buf.at[slot], sem.at[slot]
        ).start()

    for j in range(NBUF):
        start(j)

    for j in range(NCHUNK):
        slot = j % NBUF
        pltpu.make_async_copy(
            x_hbm.at[pl.ds(j * CHUNK, CHUNK), :], buf.at[slot], sem.at[slot]
        ).wait()

        logits = jnp.dot(buf[slot], wt, preferred_element_type=jnp.float32)
        logits = logits + bias

        lane_f = jax.lax.broadcasted_iota(jnp.int32, logits.shape, 1).astype(
            jnp.float32)
        m1 = jnp.max(logits, axis=1, keepdims=True)
        i1f = jnp.min(jnp.where(logits == m1, lane_f, 64.0), axis=1, keepdims=True)
        logits2 = jnp.where(lane_f == i1f, -jnp.inf, logits)
        m2 = jnp.max(logits2, axis=1, keepdims=True)
        i2f = jnp.min(jnp.where(logits2 == m2, lane_f, 64.0), axis=1, keepdims=True)

        denom = jnp.sum(jnp.exp(logits - m1), axis=1, keepdims=True)
        s1 = 1.0 / denom
        s2 = jnp.exp(m2 - m1) / denom

        scores_ref[pl.ds(j * CHUNK, CHUNK), :] = jnp.concatenate([s1, s2], axis=1)
        idx_ref[pl.ds(j * CHUNK, CHUNK), :] = jnp.concatenate(
            [i1f, i2f], axis=1).astype(jnp.int32)

        if j + NBUF < NCHUNK:
            start(j + NBUF)


@jax.jit
def kernel(x, W, b):
    wt = W.T
    b2 = b.reshape(1, N_EXPERTS)
    scores, idx = pl.pallas_call(
        _router_body,
        in_specs=[
            pl.BlockSpec(memory_space=pl.ANY),
            pl.BlockSpec((D_EMBED, N_EXPERTS), lambda: (0, 0)),
            pl.BlockSpec((1, N_EXPERTS), lambda: (0, 0)),
        ],
        out_specs=[
            pl.BlockSpec((N_TOKENS, 2), lambda: (0, 0)),
            pl.BlockSpec((N_TOKENS, 2), lambda: (0, 0)),
        ],
        out_shape=[
            jax.ShapeDtypeStruct((N_TOKENS, 2), jnp.float32),
            jax.ShapeDtypeStruct((N_TOKENS, 2), jnp.int32),
        ],
        scratch_shapes=[
            pltpu.VMEM((NBUF, CHUNK, D_EMBED), jnp.float32),
            pltpu.SemaphoreType.DMA((NBUF,)),
        ],
    )(x, wt, b2)
    return scores, idx


# speedup vs baseline: 2.6619x; 2.6619x over previous
import jax
import jax.numpy as jnp
from jax.experimental import pallas as pl

N_TOKENS = 32768
D_EMBED = 768
N_EXPERTS = 64
BLOCK = 4096


def _router_block(x_ref, wt_ref, b_ref, scores_ref, idx_ref):
    x_blk = x_ref[...]
    logits = jnp.dot(x_blk, wt_ref[...], preferred_element_type=jnp.float32)
    logits = logits + b_ref[...]
    lt = logits.T

    eid = jax.lax.broadcasted_iota(jnp.int32, lt.shape, 0).astype(jnp.float32)
    m1 = jnp.max(lt, axis=0, keepdims=True)
    i1f = jnp.min(jnp.where(lt == m1, eid, 64.0), axis=0, keepdims=True)
    lt2 = jnp.where(eid == i1f, -jnp.inf, lt)
    m2 = jnp.max(lt2, axis=0, keepdims=True)
    i2f = jnp.min(jnp.where(lt2 == m2, eid, 64.0), axis=0, keepdims=True)

    denom = jnp.sum(jnp.exp(lt - m1), axis=0, keepdims=True)
    s1 = 1.0 / denom
    s2 = jnp.exp(m2 - m1) / denom

    scores_ref[...] = jnp.concatenate([s1, s2], axis=0)
    idx_ref[...] = jnp.concatenate([i1f, i2f], axis=0).astype(jnp.int32)


@jax.jit
def kernel(x, W, b):
    wt = W.T
    b2 = b.reshape(1, N_EXPERTS)
    grid = (N_TOKENS // BLOCK,)
    scores_t, idx_t = pl.pallas_call(
        _router_block,
        grid=grid,
        in_specs=[
            pl.BlockSpec((BLOCK, D_EMBED), lambda i: (i, 0)),
            pl.BlockSpec((D_EMBED, N_EXPERTS), lambda i: (0, 0)),
            pl.BlockSpec((1, N_EXPERTS), lambda i: (0, 0)),
        ],
        out_specs=[
            pl.BlockSpec((2, BLOCK), lambda i: (0, i)),
            pl.BlockSpec((2, BLOCK), lambda i: (0, i)),
        ],
        out_shape=[
            jax.ShapeDtypeStruct((2, N_TOKENS), jnp.float32),
            jax.ShapeDtypeStruct((2, N_TOKENS), jnp.int32),
        ],
    )(x, wt, b2)
    return scores_t.T, idx_t.T
